# baseline (speedup 1.0000x reference)
import jax
import jax.numpy as jnp
from jax.experimental import pallas as pl
from jax.experimental.pallas import tpu as pltpu

_CIN = 128
_MID = 32
_COUT = 64
_H = 32
_W = 32
_HW = _H * _W

_DN0 = (((0,), (0,)), ((), ()))
_DNX = (((2,), (0,)), ((), ()))


_IPB = 4


def _fused_body(x_ref, w1_ref, b1_ref, wta_ref, w2a_ref, o_ref):
    for b in range(_IPB):
        _one_image(b, x_ref, w1_ref, b1_ref, wta_ref, w2a_ref, o_ref)


def _one_image(b, x_ref, w1_ref, b1_ref, wta_ref, w2a_ref, o_ref):
    xb = x_ref[b].astype(jnp.bfloat16).reshape(_CIN, _HW)

    a = jax.lax.dot_general(w1_ref[...], xb, _DN0,
                            preferred_element_type=jnp.float32)
    a = jnp.maximum(a + b1_ref[...], 0.0).astype(jnp.bfloat16)
    ones = jnp.ones((1, _HW), jnp.bfloat16)
    a_aug = jnp.concatenate([a, ones], axis=0)

    taps = jax.lax.dot_general(wta_ref[...], a_aug, _DN0,
                               preferred_element_type=jnp.float32)
    taps = taps.astype(jnp.bfloat16)

    def tap(t):
        return taps[t * _MID:(t + 1) * _MID]

    lane = jax.lax.broadcasted_iota(jnp.int32, (_MID, _HW), 1)
    not_last_col = (lane & (_W - 1)) != (_W - 1)
    zc = jnp.zeros((_MID, 1), jnp.bfloat16)
    zr = jnp.zeros((_MID, _W), jnp.bfloat16)
    zero = jnp.zeros((), jnp.bfloat16)

    def shw(v):
        return jnp.where(not_last_col,
                         jnp.concatenate([v[:, 1:], zc], axis=1), zero)

    def shh(v):
        return jnp.concatenate([v[:, _W:], zr], axis=1)

    c00 = tap(4)
    c01 = tap(5) + shw(tap(3))
    c10 = tap(7) + shh(tap(1))
    c11 = tap(8) + shw(tap(6)) + shh(tap(2)) + shh(shw(tap(0)))

    t_flat = jnp.concatenate(
        [jnp.maximum(c, zero) for c in (c00, c01, c10, c11)], axis=1)
    t_in = jnp.concatenate(
        [t_flat, jnp.ones((1, 4 * _HW), jnp.bfloat16)], axis=0)

    o = jax.lax.dot_general(w2a_ref[...], t_in, _DN0,
                            preferred_element_type=jnp.float32)
    o = jnp.maximum(o, 0.0).astype(jnp.bfloat16)

    ob = o.reshape(_COUT, 4 * _H, _W)
    ecol = jax.lax.broadcasted_iota(jnp.int32, (2 * _W, 2 * _W), 1)
    erow = jax.lax.broadcasted_iota(jnp.int32, (2 * _W, 2 * _W), 0)
    ecat = (ecol == 2 * (erow & (_W - 1)) + (erow // _W)).astype(jnp.bfloat16)

    ve = jnp.concatenate([ob[:, 0:_H], ob[:, _H:2 * _H]], axis=2)
    vo = jnp.concatenate([ob[:, 2 * _H:3 * _H], ob[:, 3 * _H:]], axis=2)
    o_ref[b, :, 0::2, :] = jax.lax.dot_general(
        ve, ecat, _DNX, preferred_element_type=jnp.float32)
    o_ref[b, :, 1::2, :] = jax.lax.dot_general(
        vo, ecat, _DNX, preferred_element_type=jnp.float32)


def kernel(w1, b1, wt, bt, w2, b2, x):
    n = x.shape[0]
    b1c = b1.reshape(_MID, 1)
    tap_has_bias = jnp.array([0, 0, 0, 0, 1, 1, 0, 1, 1], jnp.float32)
    btf = (tap_has_bias[:, None] * bt.reshape(1, _MID)).reshape(1, 9 * _MID)
    wta = jnp.concatenate([wt, btf.astype(jnp.bfloat16)], axis=0)
    w2a = jnp.concatenate([w2, b2.astype(jnp.bfloat16)], axis=0)

    out = pl.pallas_call(
        _fused_body,
        grid=(n // _IPB,),
        in_specs=[
            pl.BlockSpec((_IPB, _CIN, _H, _W), lambda i: (i, 0, 0, 0)),
            pl.BlockSpec((_CIN, _MID), lambda i: (0, 0)),
            pl.BlockSpec((_MID, 1), lambda i: (0, 0)),
            pl.BlockSpec((_MID + 1, 9 * _MID), lambda i: (0, 0)),
            pl.BlockSpec((_MID + 1, _COUT), lambda i: (0, 0)),
        ],
        out_specs=pl.BlockSpec((_IPB, _COUT, 2 * _H, 2 * _W),
                               lambda i: (i, 0, 0, 0)),
        out_shape=jax.ShapeDtypeStruct((n, _COUT, 2 * _H, 2 * _W),
                                       jnp.float32),
        compiler_params=pltpu.CompilerParams(
            dimension_semantics=("parallel",),
            vmem_limit_bytes=100 * 1024 * 1024,
        ),
        cost_estimate=pl.CostEstimate(
            flops=2 * n * _HW * _MID * (_CIN + 9 * _MID + 4 * _COUT),
            transcendentals=0,
            bytes_accessed=n * (_CIN * _HW * 2 + _COUT * 4 * _HW * 4),
        ),
    )(x, w1, b1c, wta, w2a)
    return out


# speedup vs baseline: 1.9576x; 1.9576x over previous
import jax
import jax.numpy as jnp
from jax.experimental import pallas as pl
from jax.experimental.pallas import tpu as pltpu

_CIN = 128
_MID = 32
_COUT = 64
_H = 32
_W = 32
_HW = _H * _W

_DN0 = (((0,), (0,)), ((), ()))
_DNX = (((2,), (0,)), ((), ()))


_IPB = 4


def _fused_body(x_ref, w1_ref, b1_ref, wta_ref, w2a_ref, o_ref):
    for b in range(_IPB):
        _one_image(b, x_ref, w1_ref, b1_ref, wta_ref, w2a_ref, o_ref)


def _one_image(b, x_ref, w1_ref, b1_ref, wta_ref, w2a_ref, o_ref):
    xb = x_ref[b]

    a = jax.lax.dot_general(w1_ref[...], xb, _DN0,
                            preferred_element_type=jnp.float32)
    a = jnp.maximum(a + b1_ref[...], 0.0).astype(jnp.bfloat16)
    ones = jnp.ones((1, _HW), jnp.bfloat16)
    a_aug = jnp.concatenate([a, ones], axis=0)

    taps = jax.lax.dot_general(wta_ref[...], a_aug, _DN0,
                               preferred_element_type=jnp.float32)
    taps = taps.astype(jnp.bfloat16)

    def tap(t):
        return taps[t * _MID:(t + 1) * _MID]

    lane = jax.lax.broadcasted_iota(jnp.int32, (_MID, _HW), 1)
    not_last_col = (lane & (_W - 1)) != (_W - 1)
    zc = jnp.zeros((_MID, 1), jnp.bfloat16)
    zr = jnp.zeros((_MID, _W), jnp.bfloat16)
    zero = jnp.zeros((), jnp.bfloat16)

    def shw(v):
        return jnp.where(not_last_col,
                         jnp.concatenate([v[:, 1:], zc], axis=1), zero)

    def shh(v):
        return jnp.concatenate([v[:, _W:], zr], axis=1)

    c00 = tap(4)
    c01 = tap(5) + shw(tap(3))
    c10 = tap(7) + shh(tap(1))
    c11 = tap(8) + shw(tap(6)) + shh(tap(2)) + shh(shw(tap(0)))

    t_flat = jnp.concatenate(
        [jnp.maximum(c, zero) for c in (c00, c01, c10, c11)], axis=1)
    t_in = jnp.concatenate(
        [t_flat, jnp.ones((1, 4 * _HW), jnp.bfloat16)], axis=0)

    o = jax.lax.dot_general(w2a_ref[...], t_in, _DN0,
                            preferred_element_type=jnp.float32)
    o = jnp.maximum(o, 0.0).astype(jnp.bfloat16)

    def pblk(i):
        return o[:, i * _HW:(i + 1) * _HW].reshape(_COUT, 8, 4 * _W)

    ve = jnp.concatenate([pblk(0), pblk(1)], axis=2)
    vo = jnp.concatenate([pblk(2), pblk(3)], axis=2)
    ecol = jax.lax.broadcasted_iota(jnp.int32, (8 * _W, 8 * _W), 1)
    erow = jax.lax.broadcasted_iota(jnp.int32, (8 * _W, 8 * _W), 0)
    e8 = (ecol == ((erow >> 5) & 3) * 64 + 2 * (erow & (_W - 1))
          + (erow >> 7)).astype(jnp.bfloat16)

    se = jax.lax.dot_general(ve, e8, _DNX,
                             preferred_element_type=jnp.float32)
    so = jax.lax.dot_general(vo, e8, _DNX,
                             preferred_element_type=jnp.float32)
    for q in range(4):
        o_ref[b, :, 2 * q::8, :] = se[:, :, q * 2 * _W:(q + 1) * 2 * _W]
        o_ref[b, :, 2 * q + 1::8, :] = so[:, :, q * 2 * _W:(q + 1) * 2 * _W]


def kernel(w1, b1, wt, bt, w2, b2, x):
    n = x.shape[0]
    x2 = x.reshape(n, _CIN, _HW).astype(jnp.bfloat16)
    b1c = b1.reshape(_MID, 1)
    tap_has_bias = jnp.array([0, 0, 0, 0, 1, 1, 0, 1, 1], jnp.float32)
    btf = (tap_has_bias[:, None] * bt.reshape(1, _MID)).reshape(1, 9 * _MID)
    wta = jnp.concatenate([wt, btf.astype(jnp.bfloat16)], axis=0)
    w2a = jnp.concatenate([w2, b2.astype(jnp.bfloat16)], axis=0)

    out = pl.pallas_call(
        _fused_body,
        grid=(n // _IPB,),
        in_specs=[
            pl.BlockSpec((_IPB, _CIN, _HW), lambda i: (i, 0, 0)),
            pl.BlockSpec((_CIN, _MID), lambda i: (0, 0)),
            pl.BlockSpec((_MID, 1), lambda i: (0, 0)),
            pl.BlockSpec((_MID + 1, 9 * _MID), lambda i: (0, 0)),
            pl.BlockSpec((_MID + 1, _COUT), lambda i: (0, 0)),
        ],
        out_specs=pl.BlockSpec((_IPB, _COUT, 2 * _H, 2 * _W),
                               lambda i: (i, 0, 0, 0)),
        out_shape=jax.ShapeDtypeStruct((n, _COUT, 2 * _H, 2 * _W),
                                       jnp.float32),
        compiler_params=pltpu.CompilerParams(
            dimension_semantics=("parallel",),
            vmem_limit_bytes=100 * 1024 * 1024,
        ),
        cost_estimate=pl.CostEstimate(
            flops=2 * n * _HW * _MID * (_CIN + 9 * _MID + 4 * _COUT),
            transcendentals=0,
            bytes_accessed=n * (_CIN * _HW * 2 + _COUT * 4 * _HW * 4),
        ),
    )(x2, w1, b1c, wta, w2a)
    return out


# speedup vs baseline: 1.9657x; 1.0042x over previous
import jax
import jax.numpy as jnp
from jax.experimental import pallas as pl
from jax.experimental.pallas import tpu as pltpu

_CIN = 128
_MID = 32
_COUT = 64
_H = 32
_W = 32
_HW = _H * _W

_DN0 = (((0,), (0,)), ((), ()))
_DNX = (((2,), (0,)), ((), ()))


_IPB = 4


def _fused_body(x_ref, w1_ref, b1_ref, wta_ref, w2a_ref, o_ref):
    for b in range(_IPB):
        _one_image(b, x_ref, w1_ref, b1_ref, wta_ref, w2a_ref, o_ref)


def _one_image(b, x_ref, w1_ref, b1_ref, wta_ref, w2a_ref, o_ref):
    xb = x_ref[b]

    a = jax.lax.dot_general(w1_ref[...], xb, _DN0,
                            preferred_element_type=jnp.float32)
    a = jnp.maximum(a + b1_ref[...], 0.0).astype(jnp.bfloat16)
    ones = jnp.ones((1, _HW), jnp.bfloat16)
    a_aug = jnp.concatenate([a, ones], axis=0)

    taps = jax.lax.dot_general(wta_ref[...], a_aug, _DN0,
                               preferred_element_type=jnp.float32)
    taps = taps.astype(jnp.bfloat16)

    def tap(t):
        return taps[t * _MID:(t + 1) * _MID]

    lane = jax.lax.broadcasted_iota(jnp.int32, (_MID, _HW), 1)
    not_last_col = (lane & (_W - 1)) != (_W - 1)
    zc = jnp.zeros((_MID, 1), jnp.bfloat16)
    zr = jnp.zeros((_MID, _W), jnp.bfloat16)
    zero = jnp.zeros((), jnp.bfloat16)

    def shw(v):
        return jnp.where(not_last_col,
                         jnp.concatenate([v[:, 1:], zc], axis=1), zero)

    def shh(v):
        return jnp.concatenate([v[:, _W:], zr], axis=1)

    c00 = tap(4)
    c01 = tap(5) + shw(tap(3))
    c10 = tap(7) + shh(tap(1))
    c11 = tap(8) + shw(tap(6)) + shh(tap(2)) + shh(shw(tap(0)))

    t_flat = jnp.concatenate(
        [jnp.maximum(c, zero) for c in (c00, c01, c10, c11)], axis=1)
    t_in = jnp.concatenate(
        [t_flat, jnp.ones((1, 4 * _HW), jnp.bfloat16)], axis=0)

    o = jax.lax.dot_general(w2a_ref[...], t_in, _DN0,
                            preferred_element_type=jnp.float32)
    o = jnp.maximum(o, 0.0).astype(jnp.bfloat16)

    def pblk(i):
        return o[:, i * _HW:(i + 1) * _HW].reshape(_COUT, 8, 4 * _W)

    ve = jnp.concatenate([pblk(0), pblk(1)], axis=2)
    vo = jnp.concatenate([pblk(2), pblk(3)], axis=2)
    ecol = jax.lax.broadcasted_iota(jnp.int32, (8 * _W, 8 * _W), 1)
    erow = jax.lax.broadcasted_iota(jnp.int32, (8 * _W, 8 * _W), 0)
    e8 = (ecol == ((erow >> 5) & 3) * 64 + 2 * (erow & (_W - 1))
          + (erow >> 7)).astype(jnp.bfloat16)

    se = jnp.dot(ve.reshape(_COUT * 8, 8 * _W), e8,
                 preferred_element_type=jnp.float32).reshape(
                     _COUT, 8, 8 * _W)
    so = jnp.dot(vo.reshape(_COUT * 8, 8 * _W), e8,
                 preferred_element_type=jnp.float32).reshape(
                     _COUT, 8, 8 * _W)
    for q in range(4):
        o_ref[b, :, 2 * q::8, :] = se[:, :, q * 2 * _W:(q + 1) * 2 * _W]
        o_ref[b, :, 2 * q + 1::8, :] = so[:, :, q * 2 * _W:(q + 1) * 2 * _W]


def kernel(w1, b1, wt, bt, w2, b2, x):
    n = x.shape[0]
    x2 = x.reshape(n, _CIN, _HW).astype(jnp.bfloat16)
    b1c = b1.reshape(_MID, 1)
    tap_has_bias = jnp.array([0, 0, 0, 0, 1, 1, 0, 1, 1], jnp.float32)
    btf = (tap_has_bias[:, None] * bt.reshape(1, _MID)).reshape(1, 9 * _MID)
    wta = jnp.concatenate([wt, btf.astype(jnp.bfloat16)], axis=0)
    w2a = jnp.concatenate([w2, b2.astype(jnp.bfloat16)], axis=0)

    out = pl.pallas_call(
        _fused_body,
        grid=(n // _IPB,),
        in_specs=[
            pl.BlockSpec((_IPB, _CIN, _HW), lambda i: (i, 0, 0)),
            pl.BlockSpec((_CIN, _MID), lambda i: (0, 0)),
            pl.BlockSpec((_MID, 1), lambda i: (0, 0)),
            pl.BlockSpec((_MID + 1, 9 * _MID), lambda i: (0, 0)),
            pl.BlockSpec((_MID + 1, _COUT), lambda i: (0, 0)),
        ],
        out_specs=pl.BlockSpec((_IPB, _COUT, 2 * _H, 2 * _W),
                               lambda i: (i, 0, 0, 0)),
        out_shape=jax.ShapeDtypeStruct((n, _COUT, 2 * _H, 2 * _W),
                                       jnp.float32),
        compiler_params=pltpu.CompilerParams(
            dimension_semantics=("parallel",),
            vmem_limit_bytes=100 * 1024 * 1024,
        ),
        cost_estimate=pl.CostEstimate(
            flops=2 * n * _HW * _MID * (_CIN + 9 * _MID + 4 * _COUT),
            transcendentals=0,
            bytes_accessed=n * (_CIN * _HW * 2 + _COUT * 4 * _HW * 4),
        ),
    )(x2, w1, b1c, wta, w2a)
    return out
